# Initial kernel scaffold; baseline (speedup 1.0000x reference)
import jax
import jax.numpy as jnp
from jax.experimental import pallas as pl


def kernel(node_embeddings, triples, change_points, device, basis, att, attention, bias):
    raise NotImplementedError("write your pallas kernel here")


# speedup vs baseline: 72.5583x; 72.5583x over previous
import functools

import jax
import jax.numpy as jnp
from jax import lax
from jax.experimental import pallas as pl
from jax.experimental.pallas import tpu as pltpu
from jax.experimental.pallas import tpu_sc as plsc

N = 10000
E = 320000
R = 8
B = 4
IN = 128
OUT = 128
SDW = 2 * R

NC = 2
NS = 16
NW = NC * NS

CH = 128
NCH = E // CH
XBLK = 1000

_BIG = 2 ** 30


def _dense_body(x_ref, basis_ref, att_ref, atmat_ref, wh_ref, sd_ref):
    xb = x_ref[...]
    bas = basis_ref[...].reshape(B, IN * OUT)
    w_all = jnp.dot(att_ref[...], bas, preferred_element_type=jnp.float32)
    parts = []
    for r in range(R):
        w = w_all[r].reshape(IN, OUT)
        wh = jnp.dot(xb, w, preferred_element_type=jnp.float32)
        wh_ref[r] = wh
        parts.append(
            jnp.dot(wh, atmat_ref[:, 2 * r:2 * r + 2],
                    preferred_element_type=jnp.float32))
    sd_ref[...] = jnp.concatenate(parts, axis=1)


_dense = pl.pallas_call(
    _dense_body,
    grid=(N // XBLK,),
    in_specs=[
        pl.BlockSpec((XBLK, IN), lambda i: (i, 0)),
        pl.BlockSpec((B, IN, OUT), lambda i: (0, 0, 0)),
        pl.BlockSpec((R, B), lambda i: (0, 0)),
        pl.BlockSpec((IN, SDW), lambda i: (0, 0)),
    ],
    out_specs=[
        pl.BlockSpec((R, XBLK, OUT), lambda i: (0, i, 0)),
        pl.BlockSpec((XBLK, SDW), lambda i: (i, 0)),
    ],
    out_shape=[
        jax.ShapeDtypeStruct((R, N, OUT), jnp.float32),
        jax.ShapeDtypeStruct((N, SDW), jnp.float32),
    ],
)


def _bridge_body(bins_ref, den_ref):
    tot = bins_ref[0] + bins_ref[1]
    for r in range(R):
        den_ref[r] = tot[:, r * 16:(r + 1) * 16]


_bridge = pl.pallas_call(
    _bridge_body,
    grid=(N // XBLK,),
    in_specs=[pl.BlockSpec((NC, XBLK, 128), lambda i: (0, i, 0))],
    out_specs=pl.BlockSpec((R, XBLK, 16), lambda i: (0, i, 0)),
    out_shape=jax.ShapeDtypeStruct((R, N, 16), jnp.float32),
)


def _combine_body(hp_ref, x_ref, bias_ref, out_ref):
    out_ref[...] = hp_ref[0] + hp_ref[1] + x_ref[...] + bias_ref[...]


_combine = pl.pallas_call(
    _combine_body,
    grid=(N // XBLK,),
    in_specs=[
        pl.BlockSpec((NC, XBLK, OUT), lambda i: (0, i, 0)),
        pl.BlockSpec((XBLK, IN), lambda i: (i, 0)),
        pl.BlockSpec((1, OUT), lambda i: (0, 0)),
    ],
    out_specs=pl.BlockSpec((XBLK, OUT), lambda i: (i, 0)),
    out_shape=jax.ShapeDtypeStruct((N, OUT), jnp.float32),
)


def _rel_of(pos, cpr):
    cnt = jnp.zeros((16,), jnp.int32)
    for j in range(R + 1):
        cnt = cnt + jnp.where(pos >= cpr[j], 1, 0)
    validf = jnp.where(cnt >= 1, 1.0, 0.0) * jnp.where(cnt <= R, 1.0, 0.0)
    rel = jnp.clip(cnt - 1, 0, R - 1)
    return rel, validf


def _epass_body(src_hbm, dst_hbm, cp_hbm, sd_hbm, e_hbm,
                src_v, dst_v, ks_v, kd_v, sv_v, dv_v, e_v, cp_v, sem):
    c = lax.axis_index("c")
    s = lax.axis_index("s")
    wid = s * NC + c
    pltpu.sync_copy(cp_hbm, cp_v)

    nch = NCH // NW + jnp.where(wid < NCH % NW, 1, 0)

    def chunk(i, carry):
        cid = wid + i * NW
        off = cid * CH
        d1 = pltpu.async_copy(src_hbm.at[pl.ds(off, CH)], src_v, sem)
        d2 = pltpu.async_copy(dst_hbm.at[pl.ds(off, CH)], dst_v, sem)
        d1.wait()
        d2.wait()
        cpr = cp_v[...]
        for g in range(CH // 16):
            sl = pl.ds(g * 16, 16)
            pos = off + g * 16 + lax.iota(jnp.int32, 16)
            rel, validf = _rel_of(pos, cpr)
            ks_v[sl] = src_v[sl] * SDW + 2 * rel
            kd_v[sl] = dst_v[sl] * SDW + 2 * rel + 1
            e_v[sl] = validf
        g1 = pltpu.async_copy(sd_hbm.at[ks_v], sv_v, sem)
        g2 = pltpu.async_copy(sd_hbm.at[kd_v], dv_v, sem)
        g1.wait()
        g2.wait()
        for g in range(CH // 16):
            sl = pl.ds(g * 16, 16)
            z = sv_v[sl] + dv_v[sl]
            z = jnp.where(z > 0.0, z, 0.2 * z)
            e_v[sl] = e_v[sl] * jnp.exp(z)
        pltpu.sync_copy(e_v, e_hbm.at[pl.ds(off, CH)])
        return carry

    lax.fori_loop(0, nch, chunk, jnp.int32(0))


def _bins_body(dst_hbm, cp_hbm, e_hbm, bins_hbm,
               dst_v, kb_v, e_v, p_v, cp_v, bins_sh, sem):
    c = lax.axis_index("c")
    s = lax.axis_index("s")
    wid = s * NC + c

    for j in range(CH):
        for t in range(8):
            p_v[j, pl.ds(t * 16, 16)] = jnp.zeros((16,), jnp.float32)
    base_z = s * 624
    for k in range(4):
        pltpu.sync_copy(p_v, bins_sh.at[pl.ds(base_z + k * CH, CH)])

    @pl.when(s < NS - 1)
    def _():
        pltpu.sync_copy(p_v.at[pl.ds(0, 112)],
                        bins_sh.at[pl.ds(base_z + 4 * CH, 112)])

    @pl.when(s == NS - 1)
    def _():
        pltpu.sync_copy(p_v, bins_sh.at[pl.ds(base_z + 4 * CH, CH)])

    pltpu.sync_copy(cp_hbm, cp_v)
    plsc.subcore_barrier()

    cpr = cp_v[...]
    for r in range(R):
        if r > 0:
            for j in range(CH):
                p_v[j, pl.ds((r - 1) * 16, 16)] = jnp.zeros(
                    (16,), jnp.float32)
        lo = cpr[r]
        hi = cpr[r + 1]
        start0 = (lo // CH) * CH
        nch = (hi - start0 + CH - 1) // CH
        mych = jnp.maximum((nch - wid + NW - 1) // NW, 0)

        def chunk(i, carry, r=r, lo=lo, hi=hi, start0=start0):
            off = start0 + (wid + i * NW) * CH
            d1 = pltpu.async_copy(dst_hbm.at[pl.ds(off, CH)], dst_v, sem)
            d2 = pltpu.async_copy(e_hbm.at[pl.ds(off, CH)], e_v, sem)
            d1.wait()
            d2.wait()
            for g in range(CH // 16):
                sl = pl.ds(g * 16, 16)
                pos = off + g * 16 + lax.iota(jnp.int32, 16)
                vf = (jnp.where(pos >= lo, 1.0, 0.0)
                      * jnp.where(pos < hi, 1.0, 0.0))
                ev = e_v[sl] * vf
                kb_v[0, sl] = dst_v[sl]
                for u in range(16):
                    p_v[g * 16 + u, pl.ds(r * 16, 16)] = jnp.broadcast_to(
                        ev[u], (16,))
            pltpu.sync_copy(p_v, bins_sh.at[kb_v.at[0]], add=True)
            return carry

        lax.fori_loop(0, mych, chunk, jnp.int32(0))

    plsc.subcore_barrier()

    @pl.when(s == 0)
    def _():
        pltpu.sync_copy(bins_sh, bins_hbm.at[c])


def _aggr_body(src_hbm, dst_hbm, cp_hbm, e_hbm, den_hbm, wh_hbm, zh_hbm,
               hp_hbm,
               src_v, dst_v, srcw_v, krow_v, kden_v, e_v, den_v,
               rows_v, cp_v, hp_sh, sem):
    c = lax.axis_index("c")
    s = lax.axis_index("s")
    wid = s * NC + c

    @pl.when(s == 0)
    def _():
        pltpu.sync_copy(zh_hbm, hp_sh)

    pltpu.sync_copy(cp_hbm, cp_v)
    plsc.subcore_barrier()

    nch = NCH // NW + jnp.where(wid < NCH % NW, 1, 0)

    def chunk(i, carry):
        cid = wid + i * NW
        off = cid * CH
        d1 = pltpu.async_copy(src_hbm.at[pl.ds(off, CH)], src_v, sem)
        d2 = pltpu.async_copy(dst_hbm.at[pl.ds(off, CH)], dst_v, sem)
        d3 = pltpu.async_copy(e_hbm.at[pl.ds(off, CH)], e_v, sem)
        d1.wait()
        d2.wait()
        d3.wait()
        cpr = cp_v[...]
        for g in range(CH // 16):
            sl = pl.ds(g * 16, 16)
            pos = off + g * 16 + lax.iota(jnp.int32, 16)
            rel, _ = _rel_of(pos, cpr)
            srcw_v[0, sl] = src_v[sl]
            krow_v[0, sl] = rel * N + dst_v[sl]
            kden_v[0, sl] = (rel * N + dst_v[sl]) * 16
        gs = [
            pltpu.async_copy(den_hbm.at[kden_v.at[0]], den_v, sem),
            pltpu.async_copy(wh_hbm.at[krow_v.at[0]], rows_v, sem),
        ]
        for d in gs:
            d.wait()
        for g in range(CH // 16):
            sl = pl.ds(g * 16, 16)
            e_v[sl] = e_v[sl] / jnp.maximum(den_v[sl], 1e-8)

        def scale(j, carry2):
            av16 = e_v[pl.ds(j * 16, 16)]
            for u in range(16):
                jj = j * 16 + u
                aj = av16[u]
                for t in range(OUT // 16):
                    sl2 = pl.ds(t * 16, 16)
                    rows_v[jj, sl2] = rows_v[jj, sl2] * aj
            return carry2

        lax.fori_loop(0, CH // 16, scale, jnp.int32(0))
        pltpu.sync_copy(rows_v, hp_sh.at[srcw_v.at[0]], add=True)
        return carry

    lax.fori_loop(0, nch, chunk, jnp.int32(0))
    plsc.subcore_barrier()

    @pl.when(s == 0)
    def _():
        pltpu.sync_copy(hp_sh, hp_hbm.at[c])


@functools.lru_cache(maxsize=None)
def _sc_kernels():
    mesh = plsc.VectorSubcoreMesh(
        core_axis_name="c", subcore_axis_name="s",
        num_cores=NC, num_subcores=NS)
    epass = pl.kernel(
        _epass_body,
        out_type=jax.ShapeDtypeStruct((E,), jnp.float32),
        mesh=mesh,
        scratch_types=[
            pltpu.VMEM((CH,), jnp.int32),
            pltpu.VMEM((CH,), jnp.int32),
            pltpu.VMEM((CH,), jnp.int32),
            pltpu.VMEM((CH,), jnp.int32),
            pltpu.VMEM((CH,), jnp.float32),
            pltpu.VMEM((CH,), jnp.float32),
            pltpu.VMEM((CH,), jnp.float32),
            pltpu.VMEM((16,), jnp.int32),
            pltpu.SemaphoreType.DMA,
        ],
    )
    bins = pl.kernel(
        _bins_body,
        out_type=jax.ShapeDtypeStruct((NC, N, 128), jnp.float32),
        mesh=mesh,
        scratch_types=[
            pltpu.VMEM((CH,), jnp.int32),
            pltpu.VMEM((1, CH), jnp.int32),
            pltpu.VMEM((CH,), jnp.float32),
            pltpu.VMEM((CH, 128), jnp.float32),
            pltpu.VMEM((16,), jnp.int32),
            pltpu.VMEM_SHARED((N, 128), jnp.float32),
            pltpu.SemaphoreType.DMA,
        ],
    )
    aggr = pl.kernel(
        _aggr_body,
        out_type=jax.ShapeDtypeStruct((NC, N, OUT), jnp.float32),
        mesh=mesh,
        scratch_types=[
            pltpu.VMEM((CH,), jnp.int32),
            pltpu.VMEM((CH,), jnp.int32),
            pltpu.VMEM((1, CH), jnp.int32),
            pltpu.VMEM((1, CH), jnp.int32),
            pltpu.VMEM((1, CH), jnp.int32),
            pltpu.VMEM((CH,), jnp.float32),
            pltpu.VMEM((CH,), jnp.float32),
            pltpu.VMEM((CH, OUT), jnp.float32),
            pltpu.VMEM((16,), jnp.int32),
            pltpu.VMEM_SHARED((N, OUT), jnp.float32),
            pltpu.SemaphoreType.DMA,
        ],
    )
    return epass, bins, aggr


def kernel(node_embeddings, triples, change_points, device, basis, att,
           attention, bias):
    del device
    src = triples[:, 0].astype(jnp.int32)
    dst = triples[:, 2].astype(jnp.int32)
    cp_pad = jnp.concatenate(
        [change_points.astype(jnp.int32), jnp.full((7,), _BIG, jnp.int32)])
    a_s = attention[:, :OUT].T
    a_d = attention[:, OUT:].T
    atmat = jnp.stack([a_s, a_d], axis=2).reshape(OUT, SDW)

    wh, sd = _dense(node_embeddings, basis, att, atmat)
    sd_flat = sd.reshape(N * SDW)
    wh_flat = wh.reshape(R * N, OUT)

    epass, bins_k, aggr = _sc_kernels()
    e = epass(src, dst, cp_pad, sd_flat)
    bins = bins_k(dst, cp_pad, e)
    dentab = _bridge(bins).reshape(R * N * 16)

    zh = jnp.zeros((N, OUT), jnp.float32)
    hp = aggr(src, dst, cp_pad, e, dentab, wh_flat, zh)

    return _combine(hp, node_embeddings, bias.reshape(1, OUT))


# speedup vs baseline: 83.6193x; 1.1524x over previous
import functools

import jax
import jax.numpy as jnp
from jax import lax
from jax.experimental import pallas as pl
from jax.experimental.pallas import tpu as pltpu
from jax.experimental.pallas import tpu_sc as plsc

N = 10000
E = 320000
R = 8
B = 4
IN = 128
OUT = 128
SDW = 2 * R

NC = 2
NS = 16
NW = NC * NS

CH = 128
NCH = E // CH
CHE = 256
NCHE = E // CHE
XBLK = 1000

_BIG = 2 ** 30


def _dense_body(x_ref, basis_ref, att_ref, atmat_ref, wh_ref, sd_ref):
    xb = x_ref[...]
    bas = basis_ref[...].reshape(B, IN * OUT)
    w_all = jnp.dot(att_ref[...], bas, preferred_element_type=jnp.float32)
    parts = []
    for r in range(R):
        w = w_all[r].reshape(IN, OUT)
        wh = jnp.dot(xb, w, preferred_element_type=jnp.float32)
        wh_ref[r] = wh
        parts.append(
            jnp.dot(wh, atmat_ref[:, 2 * r:2 * r + 2],
                    preferred_element_type=jnp.float32))
    sd_ref[...] = jnp.concatenate(parts, axis=1)


_dense = pl.pallas_call(
    _dense_body,
    grid=(N // XBLK,),
    in_specs=[
        pl.BlockSpec((XBLK, IN), lambda i: (i, 0)),
        pl.BlockSpec((B, IN, OUT), lambda i: (0, 0, 0)),
        pl.BlockSpec((R, B), lambda i: (0, 0)),
        pl.BlockSpec((IN, SDW), lambda i: (0, 0)),
    ],
    out_specs=[
        pl.BlockSpec((R, XBLK, OUT), lambda i: (0, i, 0)),
        pl.BlockSpec((XBLK, SDW), lambda i: (i, 0)),
    ],
    out_shape=[
        jax.ShapeDtypeStruct((R, N, OUT), jnp.float32),
        jax.ShapeDtypeStruct((N, SDW), jnp.float32),
    ],
)


def _bridge_body(bins_ref, den_ref):
    tot = bins_ref[0] + bins_ref[1]
    for r in range(R):
        den_ref[r] = tot[:, r * 16:(r + 1) * 16]


_bridge = pl.pallas_call(
    _bridge_body,
    grid=(N // XBLK,),
    in_specs=[pl.BlockSpec((NC, XBLK, 128), lambda i: (0, i, 0))],
    out_specs=pl.BlockSpec((R, XBLK, 16), lambda i: (0, i, 0)),
    out_shape=jax.ShapeDtypeStruct((R, N, 16), jnp.float32),
)


def _combine_body(hp_ref, x_ref, bias_ref, out_ref):
    out_ref[...] = hp_ref[0] + hp_ref[1] + x_ref[...] + bias_ref[...]


_combine = pl.pallas_call(
    _combine_body,
    grid=(N // XBLK,),
    in_specs=[
        pl.BlockSpec((NC, XBLK, OUT), lambda i: (0, i, 0)),
        pl.BlockSpec((XBLK, IN), lambda i: (i, 0)),
        pl.BlockSpec((1, OUT), lambda i: (0, 0)),
    ],
    out_specs=pl.BlockSpec((XBLK, OUT), lambda i: (i, 0)),
    out_shape=jax.ShapeDtypeStruct((N, OUT), jnp.float32),
)


def _rel_of(pos, cpr):
    cnt = jnp.zeros((16,), jnp.int32)
    for j in range(R + 1):
        cnt = cnt + jnp.where(pos >= cpr[j], 1, 0)
    validf = jnp.where(cnt >= 1, 1.0, 0.0) * jnp.where(cnt <= R, 1.0, 0.0)
    rel = jnp.clip(cnt - 1, 0, R - 1)
    return rel, validf


def _epass_body(src_hbm, dst_hbm, cp_hbm, sd_hbm, e_hbm,
                src_v, dst_v, ks_v, kd_v, sv_v, dv_v, e_v, cp_v, sem):
    c = lax.axis_index("c")
    s = lax.axis_index("s")
    wid = s * NC + c
    pltpu.sync_copy(cp_hbm, cp_v)

    nch = NCHE // NW + jnp.where(wid < NCHE % NW, 1, 0)

    def chunk(i, carry):
        cid = wid + i * NW
        off = cid * CHE
        d1 = pltpu.async_copy(src_hbm.at[pl.ds(off, CHE)], src_v, sem)
        d2 = pltpu.async_copy(dst_hbm.at[pl.ds(off, CHE)], dst_v, sem)
        d1.wait()
        d2.wait()
        cpr = cp_v[...]
        for g in range(CHE // 16):
            row, colo = g // 8, (g % 8) * 16
            sl = pl.ds(g * 16, 16)
            cs = pl.ds(colo, 16)
            pos = off + g * 16 + lax.iota(jnp.int32, 16)
            rel, validf = _rel_of(pos, cpr)
            ks_v[row, cs] = src_v[sl] * SDW + 2 * rel
            kd_v[row, cs] = dst_v[sl] * SDW + 2 * rel + 1
            e_v[sl] = validf
        gs = [
            pltpu.async_copy(sd_hbm.at[ks_v.at[0]],
                             sv_v.at[pl.ds(0, 128)], sem),
            pltpu.async_copy(sd_hbm.at[ks_v.at[1]],
                             sv_v.at[pl.ds(128, 128)], sem),
            pltpu.async_copy(sd_hbm.at[kd_v.at[0]],
                             dv_v.at[pl.ds(0, 128)], sem),
            pltpu.async_copy(sd_hbm.at[kd_v.at[1]],
                             dv_v.at[pl.ds(128, 128)], sem),
        ]
        for d in gs:
            d.wait()
        for g in range(CHE // 16):
            sl = pl.ds(g * 16, 16)
            z = sv_v[sl] + dv_v[sl]
            z = jnp.where(z > 0.0, z, 0.2 * z)
            e_v[sl] = e_v[sl] * jnp.exp(z)
        pltpu.sync_copy(e_v, e_hbm.at[pl.ds(off, CHE)])
        return carry

    lax.fori_loop(0, nch, chunk, jnp.int32(0))


def _bins_body(dst_hbm, cp_hbm, e_hbm, bins_hbm,
               dst_v, kb_v, e_v, p_v, cp_v, bins_sh, sem):
    c = lax.axis_index("c")
    s = lax.axis_index("s")
    wid = s * NC + c

    for j in range(CH):
        for t in range(8):
            p_v[j, pl.ds(t * 16, 16)] = jnp.zeros((16,), jnp.float32)
    base_z = s * 624
    for k in range(4):
        pltpu.sync_copy(p_v, bins_sh.at[pl.ds(base_z + k * CH, CH)])

    @pl.when(s < NS - 1)
    def _():
        pltpu.sync_copy(p_v.at[pl.ds(0, 112)],
                        bins_sh.at[pl.ds(base_z + 4 * CH, 112)])

    @pl.when(s == NS - 1)
    def _():
        pltpu.sync_copy(p_v, bins_sh.at[pl.ds(base_z + 4 * CH, CH)])

    pltpu.sync_copy(cp_hbm, cp_v)
    plsc.subcore_barrier()

    cpr = cp_v[...]
    for r in range(R):
        if r > 0:
            for j in range(CH):
                p_v[j, pl.ds((r - 1) * 16, 16)] = jnp.zeros(
                    (16,), jnp.float32)
        lo = cpr[r]
        hi = cpr[r + 1]
        start0 = (lo // CH) * CH
        nch = (hi - start0 + CH - 1) // CH
        mych = jnp.maximum((nch - wid + NW - 1) // NW, 0)

        def chunk(i, carry, r=r, lo=lo, hi=hi, start0=start0):
            off = start0 + (wid + i * NW) * CH
            d1 = pltpu.async_copy(dst_hbm.at[pl.ds(off, CH)], dst_v, sem)
            d2 = pltpu.async_copy(e_hbm.at[pl.ds(off, CH)], e_v, sem)
            d1.wait()
            d2.wait()
            for g in range(CH // 16):
                sl = pl.ds(g * 16, 16)
                pos = off + g * 16 + lax.iota(jnp.int32, 16)
                vf = (jnp.where(pos >= lo, 1.0, 0.0)
                      * jnp.where(pos < hi, 1.0, 0.0))
                ev = e_v[sl] * vf
                kb_v[0, sl] = dst_v[sl]
                for u in range(16):
                    p_v[g * 16 + u, pl.ds(r * 16, 16)] = jnp.broadcast_to(
                        ev[u], (16,))
            pltpu.sync_copy(p_v, bins_sh.at[kb_v.at[0]], add=True)
            return carry

        lax.fori_loop(0, mych, chunk, jnp.int32(0))

    plsc.subcore_barrier()

    @pl.when(s == 0)
    def _():
        pltpu.sync_copy(bins_sh, bins_hbm.at[c])


def _aggr_body(src_hbm, dst_hbm, cp_hbm, e_hbm, den_hbm, wh_hbm, zh_hbm,
               hp_hbm,
               src_v, dst_v, srcw_v, krow_v, kden_v, e_v, den_v,
               rows_v, cp_v, hp_sh, sem):
    c = lax.axis_index("c")
    s = lax.axis_index("s")
    wid = s * NC + c

    @pl.when(s == 0)
    def _():
        pltpu.sync_copy(zh_hbm, hp_sh)

    pltpu.sync_copy(cp_hbm, cp_v)
    plsc.subcore_barrier()

    nch = NCHE // NW + jnp.where(wid < NCHE % NW, 1, 0)

    def chunk(i, carry):
        cid = wid + i * NW
        off = cid * CHE
        d1 = pltpu.async_copy(src_hbm.at[pl.ds(off, CHE)], src_v, sem)
        d2 = pltpu.async_copy(dst_hbm.at[pl.ds(off, CHE)], dst_v, sem)
        d3 = pltpu.async_copy(e_hbm.at[pl.ds(off, CHE)], e_v, sem)
        d1.wait()
        d2.wait()
        d3.wait()
        cpr = cp_v[...]
        for g in range(CHE // 16):
            row, colo = g // 8, (g % 8) * 16
            sl = pl.ds(g * 16, 16)
            cs = pl.ds(colo, 16)
            pos = off + g * 16 + lax.iota(jnp.int32, 16)
            rel, _ = _rel_of(pos, cpr)
            srcw_v[row, cs] = src_v[sl]
            krow_v[row, cs] = rel * N + dst_v[sl]
            kden_v[row, cs] = (rel * N + dst_v[sl]) * 16
        gs = [
            pltpu.async_copy(den_hbm.at[kden_v.at[0]],
                             den_v.at[pl.ds(0, 128)], sem),
            pltpu.async_copy(den_hbm.at[kden_v.at[1]],
                             den_v.at[pl.ds(128, 128)], sem),
            pltpu.async_copy(wh_hbm.at[krow_v.at[0]],
                             rows_v.at[pl.ds(0, 128)], sem),
            pltpu.async_copy(wh_hbm.at[krow_v.at[1]],
                             rows_v.at[pl.ds(128, 128)], sem),
        ]
        for d in gs:
            d.wait()
        for g in range(CHE // 16):
            sl = pl.ds(g * 16, 16)
            e_v[sl] = e_v[sl] / jnp.maximum(den_v[sl], 1e-8)

        def scale(j, carry2):
            av16 = e_v[pl.ds(j * 16, 16)]
            for u in range(16):
                jj = j * 16 + u
                aj = av16[u]
                for t in range(OUT // 16):
                    sl2 = pl.ds(t * 16, 16)
                    rows_v[jj, sl2] = rows_v[jj, sl2] * aj
            return carry2

        lax.fori_loop(0, CHE // 16, scale, jnp.int32(0))
        pltpu.sync_copy(rows_v.at[pl.ds(0, 128)],
                        hp_sh.at[srcw_v.at[0]], add=True)
        pltpu.sync_copy(rows_v.at[pl.ds(128, 128)],
                        hp_sh.at[srcw_v.at[1]], add=True)
        return carry

    lax.fori_loop(0, nch, chunk, jnp.int32(0))
    plsc.subcore_barrier()

    @pl.when(s == 0)
    def _():
        pltpu.sync_copy(hp_sh, hp_hbm.at[c])


@functools.lru_cache(maxsize=None)
def _sc_kernels():
    mesh = plsc.VectorSubcoreMesh(
        core_axis_name="c", subcore_axis_name="s",
        num_cores=NC, num_subcores=NS)
    epass = pl.kernel(
        _epass_body,
        out_type=jax.ShapeDtypeStruct((E,), jnp.float32),
        mesh=mesh,
        scratch_types=[
            pltpu.VMEM((CHE,), jnp.int32),
            pltpu.VMEM((CHE,), jnp.int32),
            pltpu.VMEM((2, 128), jnp.int32),
            pltpu.VMEM((2, 128), jnp.int32),
            pltpu.VMEM((CHE,), jnp.float32),
            pltpu.VMEM((CHE,), jnp.float32),
            pltpu.VMEM((CHE,), jnp.float32),
            pltpu.VMEM((16,), jnp.int32),
            pltpu.SemaphoreType.DMA,
        ],
    )
    bins = pl.kernel(
        _bins_body,
        out_type=jax.ShapeDtypeStruct((NC, N, 128), jnp.float32),
        mesh=mesh,
        scratch_types=[
            pltpu.VMEM((CH,), jnp.int32),
            pltpu.VMEM((1, CH), jnp.int32),
            pltpu.VMEM((CH,), jnp.float32),
            pltpu.VMEM((CH, 128), jnp.float32),
            pltpu.VMEM((16,), jnp.int32),
            pltpu.VMEM_SHARED((N, 128), jnp.float32),
            pltpu.SemaphoreType.DMA,
        ],
    )
    aggr = pl.kernel(
        _aggr_body,
        out_type=jax.ShapeDtypeStruct((NC, N, OUT), jnp.float32),
        mesh=mesh,
        scratch_types=[
            pltpu.VMEM((CHE,), jnp.int32),
            pltpu.VMEM((CHE,), jnp.int32),
            pltpu.VMEM((2, 128), jnp.int32),
            pltpu.VMEM((2, 128), jnp.int32),
            pltpu.VMEM((2, 128), jnp.int32),
            pltpu.VMEM((CHE,), jnp.float32),
            pltpu.VMEM((CHE,), jnp.float32),
            pltpu.VMEM((CHE, OUT), jnp.float32),
            pltpu.VMEM((16,), jnp.int32),
            pltpu.VMEM_SHARED((N, OUT), jnp.float32),
            pltpu.SemaphoreType.DMA,
        ],
    )
    return epass, bins, aggr


def kernel(node_embeddings, triples, change_points, device, basis, att,
           attention, bias):
    del device
    src = triples[:, 0].astype(jnp.int32)
    dst = triples[:, 2].astype(jnp.int32)
    cp_pad = jnp.concatenate(
        [change_points.astype(jnp.int32), jnp.full((7,), _BIG, jnp.int32)])
    a_s = attention[:, :OUT].T
    a_d = attention[:, OUT:].T
    atmat = jnp.stack([a_s, a_d], axis=2).reshape(OUT, SDW)

    wh, sd = _dense(node_embeddings, basis, att, atmat)
    sd_flat = sd.reshape(N * SDW)
    wh_flat = wh.reshape(R * N, OUT)

    epass, bins_k, aggr = _sc_kernels()
    e = epass(src, dst, cp_pad, sd_flat)
    bins = bins_k(dst, cp_pad, e)
    dentab = _bridge(bins).reshape(R * N * 16)

    zh = jnp.zeros((N, OUT), jnp.float32)
    hp = aggr(src, dst, cp_pad, e, dentab, wh_flat, zh)

    return _combine(hp, node_embeddings, bias.reshape(1, OUT))


# speedup vs baseline: 83.8936x; 1.0033x over previous
import functools

import jax
import jax.numpy as jnp
from jax import lax
from jax.experimental import pallas as pl
from jax.experimental.pallas import tpu as pltpu
from jax.experimental.pallas import tpu_sc as plsc

N = 10000
E = 320000
R = 8
B = 4
IN = 128
OUT = 128
SDW = 2 * R

NC = 2
NS = 16
NW = NC * NS

CH = 128
NCH = E // CH
CHE = 256
NCHE = E // CHE
NR = CHE // 128
XBLK = 1000

_BIG = 2 ** 30


def _dense_body(x_ref, basis_ref, att_ref, atmat_ref, wh_ref, sd_ref):
    xb = x_ref[...]
    bas = basis_ref[...].reshape(B, IN * OUT)
    w_all = jnp.dot(att_ref[...], bas, preferred_element_type=jnp.float32)
    parts = []
    for r in range(R):
        w = w_all[r].reshape(IN, OUT)
        wh = jnp.dot(xb, w, preferred_element_type=jnp.float32)
        wh_ref[r] = wh
        parts.append(
            jnp.dot(wh, atmat_ref[:, 2 * r:2 * r + 2],
                    preferred_element_type=jnp.float32))
    sd_ref[...] = jnp.concatenate(parts, axis=1)


_dense = pl.pallas_call(
    _dense_body,
    grid=(N // XBLK,),
    in_specs=[
        pl.BlockSpec((XBLK, IN), lambda i: (i, 0)),
        pl.BlockSpec((B, IN, OUT), lambda i: (0, 0, 0)),
        pl.BlockSpec((R, B), lambda i: (0, 0)),
        pl.BlockSpec((IN, SDW), lambda i: (0, 0)),
    ],
    out_specs=[
        pl.BlockSpec((R, XBLK, OUT), lambda i: (0, i, 0)),
        pl.BlockSpec((XBLK, SDW), lambda i: (i, 0)),
    ],
    out_shape=[
        jax.ShapeDtypeStruct((R, N, OUT), jnp.float32),
        jax.ShapeDtypeStruct((N, SDW), jnp.float32),
    ],
)


def _bridge_body(bins_ref, den_ref):
    tot = bins_ref[0] + bins_ref[1]
    for r in range(R):
        den_ref[r] = tot[:, r * 16:(r + 1) * 16]


_bridge = pl.pallas_call(
    _bridge_body,
    grid=(N // XBLK,),
    in_specs=[pl.BlockSpec((NC, XBLK, 128), lambda i: (0, i, 0))],
    out_specs=pl.BlockSpec((R, XBLK, 16), lambda i: (0, i, 0)),
    out_shape=jax.ShapeDtypeStruct((R, N, 16), jnp.float32),
)


def _combine_body(hp_ref, x_ref, bias_ref, out_ref):
    out_ref[...] = hp_ref[0] + hp_ref[1] + x_ref[...] + bias_ref[...]


_combine = pl.pallas_call(
    _combine_body,
    grid=(N // XBLK,),
    in_specs=[
        pl.BlockSpec((NC, XBLK, OUT), lambda i: (0, i, 0)),
        pl.BlockSpec((XBLK, IN), lambda i: (i, 0)),
        pl.BlockSpec((1, OUT), lambda i: (0, 0)),
    ],
    out_specs=pl.BlockSpec((XBLK, OUT), lambda i: (i, 0)),
    out_shape=jax.ShapeDtypeStruct((N, OUT), jnp.float32),
)


def _rel_of(pos, cpr):
    cnt = jnp.zeros((16,), jnp.int32)
    for j in range(R + 1):
        cnt = cnt + jnp.where(pos >= cpr[j], 1, 0)
    validf = jnp.where(cnt >= 1, 1.0, 0.0) * jnp.where(cnt <= R, 1.0, 0.0)
    rel = jnp.clip(cnt - 1, 0, R - 1)
    return rel, validf


def _epass_body(src_hbm, dst_hbm, cp_hbm, sd_hbm, e_hbm,
                src_v, dst_v, ks_v, kd_v, sv_v, dv_v, e_v, cp_v, sem):
    c = lax.axis_index("c")
    s = lax.axis_index("s")
    wid = s * NC + c
    pltpu.sync_copy(cp_hbm, cp_v)

    nch = NCHE // NW + jnp.where(wid < NCHE % NW, 1, 0)

    def chunk(i, carry):
        cid = wid + i * NW
        off = cid * CHE
        d1 = pltpu.async_copy(src_hbm.at[pl.ds(off, CHE)], src_v, sem)
        d2 = pltpu.async_copy(dst_hbm.at[pl.ds(off, CHE)], dst_v, sem)
        d1.wait()
        d2.wait()
        cpr = cp_v[...]
        for g in range(CHE // 16):
            row, colo = g // 8, (g % 8) * 16
            sl = pl.ds(g * 16, 16)
            cs = pl.ds(colo, 16)
            pos = off + g * 16 + lax.iota(jnp.int32, 16)
            rel, validf = _rel_of(pos, cpr)
            ks_v[row, cs] = src_v[sl] * SDW + 2 * rel
            kd_v[row, cs] = dst_v[sl] * SDW + 2 * rel + 1
            e_v[sl] = validf
        gs = []
        for q in range(NR):
            gs.append(pltpu.async_copy(
                sd_hbm.at[ks_v.at[q]], sv_v.at[pl.ds(q * 128, 128)], sem))
            gs.append(pltpu.async_copy(
                sd_hbm.at[kd_v.at[q]], dv_v.at[pl.ds(q * 128, 128)], sem))
        for d in gs:
            d.wait()
        for g in range(CHE // 16):
            sl = pl.ds(g * 16, 16)
            z = sv_v[sl] + dv_v[sl]
            z = jnp.where(z > 0.0, z, 0.2 * z)
            e_v[sl] = e_v[sl] * jnp.exp(z)
        pltpu.sync_copy(e_v, e_hbm.at[pl.ds(off, CHE)])
        return carry

    lax.fori_loop(0, nch, chunk, jnp.int32(0))


def _bins_body(dst_hbm, cp_hbm, e_hbm, bins_hbm,
               dst_v, kb_v, e_v, p_v, cp_v, bins_sh, sem):
    c = lax.axis_index("c")
    s = lax.axis_index("s")
    wid = s * NC + c

    for j in range(CH):
        for t in range(8):
            p_v[j, pl.ds(t * 16, 16)] = jnp.zeros((16,), jnp.float32)
    base_z = s * 624
    for k in range(4):
        pltpu.sync_copy(p_v, bins_sh.at[pl.ds(base_z + k * CH, CH)])

    @pl.when(s < NS - 1)
    def _():
        pltpu.sync_copy(p_v.at[pl.ds(0, 112)],
                        bins_sh.at[pl.ds(base_z + 4 * CH, 112)])

    @pl.when(s == NS - 1)
    def _():
        pltpu.sync_copy(p_v, bins_sh.at[pl.ds(base_z + 4 * CH, CH)])

    pltpu.sync_copy(cp_hbm, cp_v)
    plsc.subcore_barrier()

    cpr = cp_v[...]
    for r in range(R):
        if r > 0:
            for j in range(CH):
                p_v[j, pl.ds((r - 1) * 16, 16)] = jnp.zeros(
                    (16,), jnp.float32)
        lo = cpr[r]
        hi = cpr[r + 1]
        start0 = (lo // CH) * CH
        nch = (hi - start0 + CH - 1) // CH
        mych = jnp.maximum((nch - wid + NW - 1) // NW, 0)

        def chunk(i, carry, r=r, lo=lo, hi=hi, start0=start0):
            off = start0 + (wid + i * NW) * CH
            d1 = pltpu.async_copy(dst_hbm.at[pl.ds(off, CH)], dst_v, sem)
            d2 = pltpu.async_copy(e_hbm.at[pl.ds(off, CH)], e_v, sem)
            d1.wait()
            d2.wait()
            for g in range(CH // 16):
                sl = pl.ds(g * 16, 16)
                pos = off + g * 16 + lax.iota(jnp.int32, 16)
                vf = (jnp.where(pos >= lo, 1.0, 0.0)
                      * jnp.where(pos < hi, 1.0, 0.0))
                ev = e_v[sl] * vf
                kb_v[0, sl] = dst_v[sl]
                for u in range(16):
                    p_v[g * 16 + u, pl.ds(r * 16, 16)] = jnp.broadcast_to(
                        ev[u], (16,))
            pltpu.sync_copy(p_v, bins_sh.at[kb_v.at[0]], add=True)
            return carry

        lax.fori_loop(0, mych, chunk, jnp.int32(0))

    plsc.subcore_barrier()

    @pl.when(s == 0)
    def _():
        pltpu.sync_copy(bins_sh, bins_hbm.at[c])


def _aggr_body(src_hbm, dst_hbm, cp_hbm, e_hbm, den_hbm, wh_hbm, zh_hbm,
               hp_hbm,
               src_v, dst_v, srcw_v, krow_v, kden_v, e_v, den_v,
               rows_v, cp_v, hp_sh, sem):
    c = lax.axis_index("c")
    s = lax.axis_index("s")
    wid = s * NC + c

    @pl.when(s == 0)
    def _():
        pltpu.sync_copy(zh_hbm, hp_sh)

    pltpu.sync_copy(cp_hbm, cp_v)
    plsc.subcore_barrier()

    nch = NCHE // NW + jnp.where(wid < NCHE % NW, 1, 0)

    def chunk(i, carry):
        cid = wid + i * NW
        off = cid * CHE
        d1 = pltpu.async_copy(src_hbm.at[pl.ds(off, CHE)], src_v, sem)
        d2 = pltpu.async_copy(dst_hbm.at[pl.ds(off, CHE)], dst_v, sem)
        d3 = pltpu.async_copy(e_hbm.at[pl.ds(off, CHE)], e_v, sem)
        d1.wait()
        d2.wait()
        d3.wait()
        cpr = cp_v[...]
        for g in range(CHE // 16):
            row, colo = g // 8, (g % 8) * 16
            sl = pl.ds(g * 16, 16)
            cs = pl.ds(colo, 16)
            pos = off + g * 16 + lax.iota(jnp.int32, 16)
            rel, _ = _rel_of(pos, cpr)
            srcw_v[row, cs] = src_v[sl]
            krow_v[row, cs] = rel * N + dst_v[sl]
            kden_v[row, cs] = (rel * N + dst_v[sl]) * 16
        gs = []
        for q in range(NR):
            gs.append(pltpu.async_copy(
                den_hbm.at[kden_v.at[q]], den_v.at[pl.ds(q * 128, 128)], sem))
            gs.append(pltpu.async_copy(
                wh_hbm.at[krow_v.at[q]], rows_v.at[pl.ds(q * 128, 128)], sem))
        for d in gs:
            d.wait()
        for g in range(CHE // 16):
            sl = pl.ds(g * 16, 16)
            e_v[sl] = e_v[sl] / jnp.maximum(den_v[sl], 1e-8)

        def scale(j, carry2):
            av16 = e_v[pl.ds(j * 16, 16)]
            for u in range(16):
                jj = j * 16 + u
                aj = av16[u]
                for t in range(OUT // 16):
                    sl2 = pl.ds(t * 16, 16)
                    rows_v[jj, sl2] = rows_v[jj, sl2] * aj
            return carry2

        lax.fori_loop(0, CHE // 16, scale, jnp.int32(0))
        for q in range(NR):
            pltpu.sync_copy(rows_v.at[pl.ds(q * 128, 128)],
                            hp_sh.at[srcw_v.at[q]], add=True)
        return carry

    lax.fori_loop(0, nch, chunk, jnp.int32(0))
    plsc.subcore_barrier()

    @pl.when(s == 0)
    def _():
        pltpu.sync_copy(hp_sh, hp_hbm.at[c])


@functools.lru_cache(maxsize=None)
def _sc_kernels():
    mesh = plsc.VectorSubcoreMesh(
        core_axis_name="c", subcore_axis_name="s",
        num_cores=NC, num_subcores=NS)
    epass = pl.kernel(
        _epass_body,
        out_type=jax.ShapeDtypeStruct((E,), jnp.float32),
        mesh=mesh,
        scratch_types=[
            pltpu.VMEM((CHE,), jnp.int32),
            pltpu.VMEM((CHE,), jnp.int32),
            pltpu.VMEM((CHE // 128, 128), jnp.int32),
            pltpu.VMEM((CHE // 128, 128), jnp.int32),
            pltpu.VMEM((CHE,), jnp.float32),
            pltpu.VMEM((CHE,), jnp.float32),
            pltpu.VMEM((CHE,), jnp.float32),
            pltpu.VMEM((16,), jnp.int32),
            pltpu.SemaphoreType.DMA,
        ],
    )
    bins = pl.kernel(
        _bins_body,
        out_type=jax.ShapeDtypeStruct((NC, N, 128), jnp.float32),
        mesh=mesh,
        scratch_types=[
            pltpu.VMEM((CH,), jnp.int32),
            pltpu.VMEM((1, CH), jnp.int32),
            pltpu.VMEM((CH,), jnp.float32),
            pltpu.VMEM((CH, 128), jnp.float32),
            pltpu.VMEM((16,), jnp.int32),
            pltpu.VMEM_SHARED((N, 128), jnp.float32),
            pltpu.SemaphoreType.DMA,
        ],
    )
    aggr = pl.kernel(
        _aggr_body,
        out_type=jax.ShapeDtypeStruct((NC, N, OUT), jnp.float32),
        mesh=mesh,
        scratch_types=[
            pltpu.VMEM((CHE,), jnp.int32),
            pltpu.VMEM((CHE,), jnp.int32),
            pltpu.VMEM((CHE // 128, 128), jnp.int32),
            pltpu.VMEM((CHE // 128, 128), jnp.int32),
            pltpu.VMEM((CHE // 128, 128), jnp.int32),
            pltpu.VMEM((CHE,), jnp.float32),
            pltpu.VMEM((CHE,), jnp.float32),
            pltpu.VMEM((CHE, OUT), jnp.float32),
            pltpu.VMEM((16,), jnp.int32),
            pltpu.VMEM_SHARED((N, OUT), jnp.float32),
            pltpu.SemaphoreType.DMA,
        ],
    )
    return epass, bins, aggr


def kernel(node_embeddings, triples, change_points, device, basis, att,
           attention, bias):
    del device
    src = triples[:, 0].astype(jnp.int32)
    dst = triples[:, 2].astype(jnp.int32)
    cp_pad = jnp.concatenate(
        [change_points.astype(jnp.int32), jnp.full((7,), _BIG, jnp.int32)])
    a_s = attention[:, :OUT].T
    a_d = attention[:, OUT:].T
    atmat = jnp.stack([a_s, a_d], axis=2).reshape(OUT, SDW)

    wh, sd = _dense(node_embeddings, basis, att, atmat)
    sd_flat = sd.reshape(N * SDW)
    wh_flat = wh.reshape(R * N, OUT)

    epass, bins_k, aggr = _sc_kernels()
    e = epass(src, dst, cp_pad, sd_flat)
    bins = bins_k(dst, cp_pad, e)
    dentab = _bridge(bins).reshape(R * N * 16)

    zh = jnp.zeros((N, OUT), jnp.float32)
    hp = aggr(src, dst, cp_pad, e, dentab, wh_flat, zh)

    return _combine(hp, node_embeddings, bias.reshape(1, OUT))
